# baseline (speedup 1.0000x reference)
import functools

import jax
import jax.numpy as jnp
from jax import lax
from jax.experimental import pallas as pl
from jax.experimental.pallas import tpu as pltpu
from jax.experimental.pallas import tpu_sc as plsc

N_VOCAB = 100000
N_DIM = 128
B_TOTAL = 4096 * 50
NW = 32
B_PER_W = B_TOTAL // NW
NBUF = 4
CHUNK = 200
N_CHUNKS = B_PER_W // CHUNK
N_OUTER = N_CHUNKS // NBUF


def _make_emb_kernel():
    mesh = plsc.VectorSubcoreMesh(core_axis_name="c", subcore_axis_name="s")

    scratch = [pltpu.VMEM((B_PER_W,), jnp.int32)]
    scratch += [pltpu.VMEM((CHUNK, N_DIM), jnp.float32) for _ in range(NBUF)]
    scratch += [pltpu.SemaphoreType.DMA for _ in range(2 * NBUF)]

    @functools.partial(
        pl.kernel,
        mesh=mesh,
        out_type=jax.ShapeDtypeStruct((B_TOTAL, N_DIM), jnp.float32),
        scratch_types=scratch,
    )
    def emb(weight_hbm, idx_hbm, out_hbm, idx_v, *bufs_and_sems):
        rows = bufs_and_sems[:NBUF]
        gsem = bufs_and_sems[NBUF:2 * NBUF]
        ssem = bufs_and_sems[2 * NBUF:]

        wid = lax.axis_index("s") * 2 + lax.axis_index("c")
        base = wid * B_PER_W
        pltpu.sync_copy(idx_hbm.at[pl.ds(base, B_PER_W)], idx_v)

        def issue_gather(c, b):
            pltpu.async_copy(
                weight_hbm.at[idx_v.at[pl.ds(c * CHUNK, CHUNK)]],
                rows[b],
                gsem[b],
            )

        def wait_gather(c, b):
            pltpu.make_async_copy(
                weight_hbm.at[idx_v.at[pl.ds(c * CHUNK, CHUNK)]],
                rows[b],
                gsem[b],
            ).wait()

        def issue_scatter(c, b):
            pltpu.async_copy(
                rows[b],
                out_hbm.at[pl.ds(base + c * CHUNK, CHUNK)],
                ssem[b],
            )

        def wait_scatter(c, b):
            pltpu.make_async_copy(
                rows[b],
                out_hbm.at[pl.ds(base + c * CHUNK, CHUNK)],
                ssem[b],
            ).wait()

        for b in range(NBUF):
            issue_gather(b, b)

        def outer(o):
            for b in range(NBUF):
                c = o * NBUF + b
                wait_gather(c, b)
                issue_scatter(c, b)
            for b in range(NBUF):
                c = o * NBUF + b
                wait_scatter(c, b)
                issue_gather(c + NBUF, b)

        pl.loop(0, N_OUTER - 1)(outer)

        for b in range(NBUF):
            c = (N_OUTER - 1) * NBUF + b
            wait_gather(c, b)
            issue_scatter(c, b)
        for b in range(NBUF):
            c = (N_OUTER - 1) * NBUF + b
            wait_scatter(c, b)


    return emb


_emb = _make_emb_kernel()


@jax.jit
def kernel(input, weight):
    idx = input.astype(jnp.int32).T.reshape(-1)
    out = _emb(weight, idx)
    return out.reshape(50, 4096, N_DIM).transpose(1, 0, 2)


# speedup vs baseline: 1.0336x; 1.0336x over previous
import functools

import jax
import jax.numpy as jnp
from jax import lax
from jax.experimental import pallas as pl
from jax.experimental.pallas import tpu as pltpu
from jax.experimental.pallas import tpu_sc as plsc

N_VOCAB = 100000
N_DIM = 128
B_TOTAL = 4096 * 50
NW = 32
B_PER_W = B_TOTAL // NW
NBUF = 5
CHUNK = 128
N_CHUNKS = B_PER_W // CHUNK
N_OUTER = N_CHUNKS // NBUF


def _make_emb_kernel():
    mesh = plsc.VectorSubcoreMesh(core_axis_name="c", subcore_axis_name="s")

    scratch = [pltpu.VMEM((B_PER_W,), jnp.int32)]
    scratch += [pltpu.VMEM((CHUNK, N_DIM), jnp.float32) for _ in range(NBUF)]
    scratch += [pltpu.SemaphoreType.DMA for _ in range(2 * NBUF)]

    @functools.partial(
        pl.kernel,
        mesh=mesh,
        out_type=jax.ShapeDtypeStruct((B_TOTAL, N_DIM), jnp.float32),
        scratch_types=scratch,
    )
    def emb(weight_hbm, idx_hbm, out_hbm, idx_v, *bufs_and_sems):
        rows = bufs_and_sems[:NBUF]
        gsem = bufs_and_sems[NBUF:2 * NBUF]
        ssem = bufs_and_sems[2 * NBUF:]

        wid = lax.axis_index("s") * 2 + lax.axis_index("c")
        base = wid * B_PER_W
        pltpu.sync_copy(idx_hbm.at[pl.ds(base, B_PER_W)], idx_v)

        def issue_gather(c, b):
            pltpu.async_copy(
                weight_hbm.at[idx_v.at[pl.ds(c * CHUNK, CHUNK)]],
                rows[b],
                gsem[b],
            )

        def wait_gather(c, b):
            pltpu.make_async_copy(
                weight_hbm.at[idx_v.at[pl.ds(c * CHUNK, CHUNK)]],
                rows[b],
                gsem[b],
            ).wait()

        def issue_scatter(c, b):
            pltpu.async_copy(
                rows[b],
                out_hbm.at[pl.ds(base + c * CHUNK, CHUNK)],
                ssem[b],
            )

        def wait_scatter(c, b):
            pltpu.make_async_copy(
                rows[b],
                out_hbm.at[pl.ds(base + c * CHUNK, CHUNK)],
                ssem[b],
            ).wait()

        for b in range(NBUF):
            issue_gather(b, b)

        def outer(o):
            for b in range(NBUF):
                c = o * NBUF + b
                wait_gather(c, b)
                issue_scatter(c, b)
            for b in range(NBUF):
                c = o * NBUF + b
                wait_scatter(c, b)
                issue_gather(c + NBUF, b)

        pl.loop(0, N_OUTER - 1)(outer)

        for b in range(NBUF):
            c = (N_OUTER - 1) * NBUF + b
            wait_gather(c, b)
            issue_scatter(c, b)
        for b in range(NBUF):
            c = (N_OUTER - 1) * NBUF + b
            wait_scatter(c, b)


    return emb


_emb = _make_emb_kernel()


@jax.jit
def kernel(input, weight):
    idx = input.astype(jnp.int32).T.reshape(-1)
    out = _emb(weight, idx)
    return out.reshape(50, 4096, N_DIM).transpose(1, 0, 2)


# speedup vs baseline: 1.0408x; 1.0070x over previous
import functools

import jax
import jax.numpy as jnp
from jax import lax
from jax.experimental import pallas as pl
from jax.experimental.pallas import tpu as pltpu
from jax.experimental.pallas import tpu_sc as plsc

N_VOCAB = 100000
N_DIM = 128
B_TOTAL = 4096 * 50
NW = 32
B_PER_W = B_TOTAL // NW
NBUF = 10
CHUNK = 64
N_CHUNKS = B_PER_W // CHUNK
N_OUTER = N_CHUNKS // NBUF


def _make_emb_kernel():
    mesh = plsc.VectorSubcoreMesh(core_axis_name="c", subcore_axis_name="s")

    scratch = [pltpu.VMEM((B_PER_W,), jnp.int32)]
    scratch += [pltpu.VMEM((CHUNK, N_DIM), jnp.float32) for _ in range(NBUF)]
    scratch += [pltpu.SemaphoreType.DMA for _ in range(2 * NBUF)]

    @functools.partial(
        pl.kernel,
        mesh=mesh,
        out_type=jax.ShapeDtypeStruct((B_TOTAL, N_DIM), jnp.float32),
        scratch_types=scratch,
    )
    def emb(weight_hbm, idx_hbm, out_hbm, idx_v, *bufs_and_sems):
        rows = bufs_and_sems[:NBUF]
        gsem = bufs_and_sems[NBUF:2 * NBUF]
        ssem = bufs_and_sems[2 * NBUF:]

        wid = lax.axis_index("s") * 2 + lax.axis_index("c")
        base = wid * B_PER_W
        pltpu.sync_copy(idx_hbm.at[pl.ds(base, B_PER_W)], idx_v)

        def issue_gather(c, b):
            pltpu.async_copy(
                weight_hbm.at[idx_v.at[pl.ds(c * CHUNK, CHUNK)]],
                rows[b],
                gsem[b],
            )

        def wait_gather(c, b):
            pltpu.make_async_copy(
                weight_hbm.at[idx_v.at[pl.ds(c * CHUNK, CHUNK)]],
                rows[b],
                gsem[b],
            ).wait()

        def issue_scatter(c, b):
            pltpu.async_copy(
                rows[b],
                out_hbm.at[pl.ds(base + c * CHUNK, CHUNK)],
                ssem[b],
            )

        def wait_scatter(c, b):
            pltpu.make_async_copy(
                rows[b],
                out_hbm.at[pl.ds(base + c * CHUNK, CHUNK)],
                ssem[b],
            ).wait()

        for b in range(NBUF):
            issue_gather(b, b)

        def outer(o):
            for b in range(NBUF):
                c = o * NBUF + b
                wait_gather(c, b)
                issue_scatter(c, b)
            for b in range(NBUF):
                c = o * NBUF + b
                wait_scatter(c, b)
                issue_gather(c + NBUF, b)

        pl.loop(0, N_OUTER - 1)(outer)

        for b in range(NBUF):
            c = (N_OUTER - 1) * NBUF + b
            wait_gather(c, b)
            issue_scatter(c, b)
        for b in range(NBUF):
            c = (N_OUTER - 1) * NBUF + b
            wait_scatter(c, b)


    return emb


_emb = _make_emb_kernel()


@jax.jit
def kernel(input, weight):
    idx = input.astype(jnp.int32).T.reshape(-1)
    out = _emb(weight, idx)
    return out.reshape(50, 4096, N_DIM).transpose(1, 0, 2)


# speedup vs baseline: 1.0506x; 1.0094x over previous
import functools

import jax
import jax.numpy as jnp
from jax import lax
from jax.experimental import pallas as pl
from jax.experimental.pallas import tpu as pltpu
from jax.experimental.pallas import tpu_sc as plsc

N_VOCAB = 100000
N_DIM = 128
B_TOTAL = 4096 * 50
NW = 32
B_PER_W = B_TOTAL // NW
NBUF = 8
CHUNK = 40
N_CHUNKS = B_PER_W // CHUNK
N_OUTER = N_CHUNKS // NBUF


def _make_emb_kernel():
    mesh = plsc.VectorSubcoreMesh(core_axis_name="c", subcore_axis_name="s")

    scratch = [pltpu.VMEM((B_PER_W,), jnp.int32)]
    scratch += [pltpu.VMEM((CHUNK, N_DIM), jnp.float32) for _ in range(NBUF)]
    scratch += [pltpu.SemaphoreType.DMA for _ in range(2 * NBUF)]

    @functools.partial(
        pl.kernel,
        mesh=mesh,
        out_type=jax.ShapeDtypeStruct((B_TOTAL, N_DIM), jnp.float32),
        scratch_types=scratch,
    )
    def emb(weight_hbm, idx_hbm, out_hbm, idx_v, *bufs_and_sems):
        rows = bufs_and_sems[:NBUF]
        gsem = bufs_and_sems[NBUF:2 * NBUF]
        ssem = bufs_and_sems[2 * NBUF:]

        wid = lax.axis_index("s") * 2 + lax.axis_index("c")
        base = wid * B_PER_W
        pltpu.sync_copy(idx_hbm.at[pl.ds(base, B_PER_W)], idx_v)

        def issue_gather(c, b):
            pltpu.async_copy(
                weight_hbm.at[idx_v.at[pl.ds(c * CHUNK, CHUNK)]],
                rows[b],
                gsem[b],
            )

        def wait_gather(c, b):
            pltpu.make_async_copy(
                weight_hbm.at[idx_v.at[pl.ds(c * CHUNK, CHUNK)]],
                rows[b],
                gsem[b],
            ).wait()

        def issue_scatter(c, b):
            pltpu.async_copy(
                rows[b],
                out_hbm.at[pl.ds(base + c * CHUNK, CHUNK)],
                ssem[b],
            )

        def wait_scatter(c, b):
            pltpu.make_async_copy(
                rows[b],
                out_hbm.at[pl.ds(base + c * CHUNK, CHUNK)],
                ssem[b],
            ).wait()

        for b in range(NBUF):
            issue_gather(b, b)

        def outer(o):
            for b in range(NBUF):
                c = o * NBUF + b
                wait_gather(c, b)
                issue_scatter(c, b)
            for b in range(NBUF):
                c = o * NBUF + b
                wait_scatter(c, b)
                issue_gather(c + NBUF, b)

        pl.loop(0, N_OUTER - 1)(outer)

        for b in range(NBUF):
            c = (N_OUTER - 1) * NBUF + b
            wait_gather(c, b)
            issue_scatter(c, b)
        for b in range(NBUF):
            c = (N_OUTER - 1) * NBUF + b
            wait_scatter(c, b)


    return emb


_emb = _make_emb_kernel()


@jax.jit
def kernel(input, weight):
    idx = input.astype(jnp.int32).T.reshape(-1)
    out = _emb(weight, idx)
    return out.reshape(50, 4096, N_DIM).transpose(1, 0, 2)
